# baseline (speedup 1.0000x reference)
import functools

import jax
import jax.numpy as jnp
from jax import lax
from jax.experimental import pallas as pl
from jax.experimental.pallas import tpu as pltpu
from jax.experimental.pallas import tpu_sc as plsc

B = 16384
D = 32
L = 16
NW = 32
C = B // NW
TCH = 64
NCH = C // TCH
N = 1000000
NMAIN = 999936
GMAIN = NMAIN // 4
MARGIN = 1.0
EPS = 1e-6

_mesh = plsc.VectorSubcoreMesh(core_axis_name="c", subcore_axis_name="s")


def _gidx_map(iv):
    g = ((iv >> 9) << 6) | ((iv & 127) >> 1)
    return jnp.minimum(g, GMAIN // 2 - 1)


def _pick(buf, tv, iv_chunk, k, r):
    idx_s = iv_chunk[k]
    o16 = (idx_s & 1) * 64 + ((idx_s >> 7) & 3) * L
    e_t = idx_s - NMAIN
    tr = jnp.clip(e_t >> 2, 0, 15)
    q32t = (e_t & 3) * D
    w = jnp.where(idx_s >= NMAIN, 1.0, 0.0)
    vi = buf[r, pl.ds(o16, L)]
    m0 = lax.bitcast_convert_type(vi << 16, jnp.float32)
    m1 = lax.bitcast_convert_type((vi >> 16) << 16, jnp.float32)
    t0 = tv[tr, pl.ds(q32t, L)]
    t1 = tv[tr, pl.ds(q32t + L, L)]
    return m0 + w * (t0 - m0), m1 + w * (t1 - m1)


def _u_body(ent_hbm, etail_hbm, idx_hbm, uout_hbm,
            iph, ipt, inh, int_,
            gph, gpt, gnh, gnt,
            bph, bpt, bnh, bnt,
            etv, upv, unv, sem):
    cid = lax.axis_index("c")
    sid = lax.axis_index("s")
    base = (sid * 2 + cid) * C

    ivs = (iph, ipt, inh, int_)
    gvs = (gph, gpt, gnh, gnt)
    bufs = (bph, bpt, bnh, bnt)
    segs = (0, 1, 3, 4)

    pltpu.sync_copy(etail_hbm, etv)
    for i, seg in enumerate(segs):
        pltpu.sync_copy(idx_hbm.at[pl.ds(seg * B + base, C)], ivs[i])

    def shift_body(j, carry):
        for i in range(4):
            gvs[i][pl.ds(j * L, L)] = _gidx_map(ivs[i][pl.ds(j * L, L)])
        return carry
    lax.fori_loop(0, C // L, shift_body, 0)

    def chunk_body(ch, carry):
        cbase = ch * TCH
        cps = [
            pltpu.async_copy(ent_hbm.at[gvs[i].at[pl.ds(cbase, TCH)]],
                             bufs[i], sem)
            for i in range(4)
        ]
        for cp in cps:
            cp.wait()

        for g in range(TCH // L):
            gr0 = cbase + g * L
            ivc = [ivs[i][pl.ds(gr0, L)] for i in range(4)]
            for k in range(L):
                r = g * L + k
                h0, h1 = _pick(bph, etv, ivc[0], k, r)
                t0, t1 = _pick(bpt, etv, ivc[1], k, r)
                upv[pl.ds((gr0 + k) * D, L)] = h0 - t0
                upv[pl.ds((gr0 + k) * D + L, L)] = h1 - t1
                h0, h1 = _pick(bnh, etv, ivc[2], k, r)
                t0, t1 = _pick(bnt, etv, ivc[3], k, r)
                unv[pl.ds((gr0 + k) * D, L)] = h0 - t0
                unv[pl.ds((gr0 + k) * D + L, L)] = h1 - t1
        return carry

    lax.fori_loop(0, NCH, chunk_body, 0)

    pltpu.sync_copy(upv, uout_hbm.at[pl.ds(base * D, C * D)])
    pltpu.sync_copy(unv, uout_hbm.at[pl.ds(B * D + base * D, C * D)])


_sc_u = functools.partial(
    pl.kernel,
    out_type=jax.ShapeDtypeStruct((2 * B * D,), jnp.float32),
    mesh=_mesh,
    scratch_types=(
        [pltpu.VMEM((C,), jnp.int32) for _ in range(4)]
        + [pltpu.VMEM((C,), jnp.int32) for _ in range(4)]
        + [pltpu.VMEM((TCH, 128), jnp.int32) for _ in range(4)]
        + [pltpu.VMEM((16, 128), jnp.float32),
           pltpu.VMEM((C * D,), jnp.float32),
           pltpu.VMEM((C * D,), jnp.float32),
           pltpu.SemaphoreType.DMA]
    ),
)(_u_body)


def _sq_body(rel_hbm, rtail_hbm, idx_hbm, u_hbm, out_hbm,
             ipr, inr, gpr, gnr, bpr, bnr,
             rtv, upv, unv, sqp, sqn, sem):
    cid = lax.axis_index("c")
    sid = lax.axis_index("s")
    base = (sid * 2 + cid) * C

    pltpu.sync_copy(rtail_hbm, rtv)
    pltpu.sync_copy(idx_hbm.at[pl.ds(2 * B + base, C)], ipr)
    pltpu.sync_copy(idx_hbm.at[pl.ds(5 * B + base, C)], inr)
    pltpu.sync_copy(u_hbm.at[pl.ds(base * D, C * D)], upv)
    pltpu.sync_copy(u_hbm.at[pl.ds(B * D + base * D, C * D)], unv)

    def shift_body(j, carry):
        gpr[pl.ds(j * L, L)] = _gidx_map(ipr[pl.ds(j * L, L)])
        gnr[pl.ds(j * L, L)] = _gidx_map(inr[pl.ds(j * L, L)])
        return carry
    lax.fori_loop(0, C // L, shift_body, 0)

    def chunk_body(ch, carry):
        cbase = ch * TCH
        cps = [
            pltpu.async_copy(rel_hbm.at[gpr.at[pl.ds(cbase, TCH)]], bpr, sem),
            pltpu.async_copy(rel_hbm.at[gnr.at[pl.ds(cbase, TCH)]], bnr, sem),
        ]
        for cp in cps:
            cp.wait()

        for g in range(TCH // L):
            gr0 = cbase + g * L
            ivcp = ipr[pl.ds(gr0, L)]
            ivcn = inr[pl.ds(gr0, L)]
            for k in range(L):
                r = g * L + k
                r0, r1 = _pick(bpr, rtv, ivcp, k, r)
                d0 = upv[pl.ds((gr0 + k) * D, L)] + r0 + EPS
                d1 = upv[pl.ds((gr0 + k) * D + L, L)] + r1 + EPS
                sqp[pl.ds((gr0 + k) * L, L)] = d0 * d0 + d1 * d1
                r0, r1 = _pick(bnr, rtv, ivcn, k, r)
                d0 = unv[pl.ds((gr0 + k) * D, L)] + r0 + EPS
                d1 = unv[pl.ds((gr0 + k) * D + L, L)] + r1 + EPS
                sqn[pl.ds((gr0 + k) * L, L)] = d0 * d0 + d1 * d1
        return carry

    lax.fori_loop(0, NCH, chunk_body, 0)

    pltpu.sync_copy(sqp, out_hbm.at[pl.ds(base * L, C * L)])
    pltpu.sync_copy(sqn, out_hbm.at[pl.ds(B * L + base * L, C * L)])


_sc_sq = functools.partial(
    pl.kernel,
    out_type=jax.ShapeDtypeStruct((2 * B * L,), jnp.float32),
    mesh=_mesh,
    scratch_types=(
        [pltpu.VMEM((C,), jnp.int32) for _ in range(2)]
        + [pltpu.VMEM((C,), jnp.int32) for _ in range(2)]
        + [pltpu.VMEM((TCH, 128), jnp.int32) for _ in range(2)]
        + [pltpu.VMEM((16, 128), jnp.float32),
           pltpu.VMEM((C * D,), jnp.float32),
           pltpu.VMEM((C * D,), jnp.float32),
           pltpu.VMEM((C * L,), jnp.float32),
           pltpu.VMEM((C * L,), jnp.float32),
           pltpu.SemaphoreType.DMA]
    ),
)(_sq_body)

_W = 15872
_NBLK = NMAIN // _W


def _t_body(x_ref, o_ref):
    y = jnp.transpose(x_ref[...])
    y5 = y.reshape(_W // 512, 4, 64, 2, D)
    for q in range(4):
        for par in range(2):
            blk = y5[:, q, :, par, :].reshape(_W // 8, D)
            xb = lax.bitcast_convert_type(blk, jnp.uint32)
            rb = (xb + 0x7FFF + ((xb >> 16) & 1)) >> 16
            packed = rb[:, :L] | (rb[:, L:] << 16)
            o_ref[:, par * 64 + q * L: par * 64 + (q + 1) * L] = (
                lax.bitcast_convert_type(packed, jnp.int32))


_to_rows = pl.pallas_call(
    _t_body,
    grid=(_NBLK,),
    in_specs=[pl.BlockSpec((32, _W), lambda j: (0, j))],
    out_specs=pl.BlockSpec((_W // 8, 128), lambda j: (j, 0)),
    out_shape=jax.ShapeDtypeStruct((GMAIN // 2, 128), jnp.int32),
)

_ROWS = 2 * B * L // 128
_HALF = _ROWS // 2


def _finish_body(x_ref, o_ref):
    x = x_ref[...]
    i128 = lax.broadcasted_iota(jnp.int32, (128, 8), 0)
    i8 = lax.broadcasted_iota(jnp.int32, (128, 8), 1)
    s_mat = jnp.where(i128 // L == i8, 1.0, 0.0).astype(jnp.float32)
    d2p = jnp.dot(x[:_HALF], s_mat, preferred_element_type=jnp.float32)
    d2n = jnp.dot(x[_HALF:], s_mat, preferred_element_type=jnp.float32)
    m = jnp.sqrt(d2p) - jnp.sqrt(d2n) + MARGIN
    o_ref[...] = jnp.sum(jnp.maximum(m, 0.0), keepdims=True) * (1.0 / B)


_finish = pl.pallas_call(
    _finish_body,
    out_shape=jax.ShapeDtypeStruct((1, 1), jnp.float32),
)


def kernel(pos_x, neg_x, entity_weight, relation_weight):
    pos = pos_x.astype(jnp.int32)
    neg = neg_x.astype(jnp.int32)
    idx_flat = jnp.concatenate([
        pos[:, 0], pos[:, 2], pos[:, 1],
        neg[:, 0], neg[:, 2], neg[:, 1],
    ])
    ent4 = _to_rows(entity_weight.T)
    etail = entity_weight[NMAIN:].reshape(16, 128)
    u = _sc_u(ent4, etail, idx_flat)
    rel4 = _to_rows(relation_weight.T)
    rtail = relation_weight[NMAIN:].reshape(16, 128)
    sq = _sc_sq(rel4, rtail, idx_flat, u)
    return _finish(sq.reshape(_ROWS, 128))[0, 0]


# speedup vs baseline: 3.9200x; 3.9200x over previous
import functools

import jax
import jax.numpy as jnp
from jax import lax
from jax.experimental import pallas as pl
from jax.experimental.pallas import tpu as pltpu
from jax.experimental.pallas import tpu_sc as plsc

B = 16384
D = 32
L = 16
NW = 32
C = B // NW
TCH = 128
NCH = C // TCH
N = 1000000
NMAIN = 999936
GMAIN = NMAIN // 4
MARGIN = 1.0
EPS = 1e-6

_mesh = plsc.VectorSubcoreMesh(core_axis_name="c", subcore_axis_name="s")


def _gidx_map(iv):
    g = ((iv >> 9) << 7) | (iv & 127)
    return jnp.minimum(g, GMAIN - 1)


def _pick(buf, tv, iv_chunk, k, r):
    idx_s = iv_chunk[k]
    q32m = ((idx_s >> 7) & 3) * D
    e_t = idx_s - NMAIN
    tr = jnp.clip(e_t >> 2, 0, 15)
    q32t = (e_t & 3) * D
    w = jnp.where(idx_s >= NMAIN, 1.0, 0.0)
    m0 = buf[r, pl.ds(q32m, L)]
    m1 = buf[r, pl.ds(q32m + L, L)]
    t0 = tv[tr, pl.ds(q32t, L)]
    t1 = tv[tr, pl.ds(q32t + L, L)]
    return m0 + w * (t0 - m0), m1 + w * (t1 - m1)


def _u_body(ent_hbm, etail_hbm, idx_hbm, uout_hbm,
            iph, ipt, inh, int_,
            gph, gpt, gnh, gnt,
            bph, bpt, bnh, bnt,
            etv, upv, unv, sem):
    cid = lax.axis_index("c")
    sid = lax.axis_index("s")
    base = (sid * 2 + cid) * C

    ivs = (iph, ipt, inh, int_)
    gvs = (gph, gpt, gnh, gnt)
    bufs = (bph, bpt, bnh, bnt)
    segs = (0, 1, 3, 4)

    pltpu.sync_copy(etail_hbm, etv)
    for i, seg in enumerate(segs):
        pltpu.sync_copy(idx_hbm.at[pl.ds(seg * B + base, C)], ivs[i])

    def shift_body(j, carry):
        for i in range(4):
            gvs[i][pl.ds(j * L, L)] = _gidx_map(ivs[i][pl.ds(j * L, L)])
        return carry
    lax.fori_loop(0, C // L, shift_body, 0)

    def chunk_body(ch, carry):
        cbase = ch * TCH
        cps = [
            pltpu.async_copy(ent_hbm.at[gvs[i].at[pl.ds(cbase, TCH)]],
                             bufs[i], sem)
            for i in range(4)
        ]
        for cp in cps:
            cp.wait()

        def group_body(g, carry2):
            gr0 = cbase + g * L
            ivc = [ivs[i][pl.ds(gr0, L)] for i in range(4)]
            for k in range(L):
                r = g * L + k
                h0, h1 = _pick(bph, etv, ivc[0], k, r)
                t0, t1 = _pick(bpt, etv, ivc[1], k, r)
                upv[pl.ds((gr0 + k) * D, L)] = h0 - t0
                upv[pl.ds((gr0 + k) * D + L, L)] = h1 - t1
                h0, h1 = _pick(bnh, etv, ivc[2], k, r)
                t0, t1 = _pick(bnt, etv, ivc[3], k, r)
                unv[pl.ds((gr0 + k) * D, L)] = h0 - t0
                unv[pl.ds((gr0 + k) * D + L, L)] = h1 - t1
            return carry2

        lax.fori_loop(0, TCH // L, group_body, 0)
        return carry

    lax.fori_loop(0, NCH, chunk_body, 0)

    pltpu.sync_copy(upv, uout_hbm.at[pl.ds(base * D, C * D)])
    pltpu.sync_copy(unv, uout_hbm.at[pl.ds(B * D + base * D, C * D)])


_sc_u = functools.partial(
    pl.kernel,
    out_type=jax.ShapeDtypeStruct((2 * B * D,), jnp.float32),
    mesh=_mesh,
    scratch_types=(
        [pltpu.VMEM((C,), jnp.int32) for _ in range(4)]
        + [pltpu.VMEM((C,), jnp.int32) for _ in range(4)]
        + [pltpu.VMEM((TCH, 128), jnp.float32) for _ in range(4)]
        + [pltpu.VMEM((16, 128), jnp.float32),
           pltpu.VMEM((C * D,), jnp.float32),
           pltpu.VMEM((C * D,), jnp.float32),
           pltpu.SemaphoreType.DMA]
    ),
)(_u_body)


def _sq_body(rel_hbm, rtail_hbm, idx_hbm, u_hbm, out_hbm,
             ipr, inr, gpr, gnr, bpr, bnr,
             rtv, upv, unv, sqp, sqn, sem):
    cid = lax.axis_index("c")
    sid = lax.axis_index("s")
    base = (sid * 2 + cid) * C

    pltpu.sync_copy(rtail_hbm, rtv)
    pltpu.sync_copy(idx_hbm.at[pl.ds(2 * B + base, C)], ipr)
    pltpu.sync_copy(idx_hbm.at[pl.ds(5 * B + base, C)], inr)
    pltpu.sync_copy(u_hbm.at[pl.ds(base * D, C * D)], upv)
    pltpu.sync_copy(u_hbm.at[pl.ds(B * D + base * D, C * D)], unv)

    def shift_body(j, carry):
        gpr[pl.ds(j * L, L)] = _gidx_map(ipr[pl.ds(j * L, L)])
        gnr[pl.ds(j * L, L)] = _gidx_map(inr[pl.ds(j * L, L)])
        return carry
    lax.fori_loop(0, C // L, shift_body, 0)

    def chunk_body(ch, carry):
        cbase = ch * TCH
        cps = [
            pltpu.async_copy(rel_hbm.at[gpr.at[pl.ds(cbase, TCH)]], bpr, sem),
            pltpu.async_copy(rel_hbm.at[gnr.at[pl.ds(cbase, TCH)]], bnr, sem),
        ]
        for cp in cps:
            cp.wait()

        def group_body(g, carry2):
            gr0 = cbase + g * L
            ivcp = ipr[pl.ds(gr0, L)]
            ivcn = inr[pl.ds(gr0, L)]
            for k in range(L):
                r = g * L + k
                r0, r1 = _pick(bpr, rtv, ivcp, k, r)
                d0 = upv[pl.ds((gr0 + k) * D, L)] + r0 + EPS
                d1 = upv[pl.ds((gr0 + k) * D + L, L)] + r1 + EPS
                sqp[pl.ds((gr0 + k) * L, L)] = d0 * d0 + d1 * d1
                r0, r1 = _pick(bnr, rtv, ivcn, k, r)
                d0 = unv[pl.ds((gr0 + k) * D, L)] + r0 + EPS
                d1 = unv[pl.ds((gr0 + k) * D + L, L)] + r1 + EPS
                sqn[pl.ds((gr0 + k) * L, L)] = d0 * d0 + d1 * d1
            return carry2

        lax.fori_loop(0, TCH // L, group_body, 0)
        return carry

    lax.fori_loop(0, NCH, chunk_body, 0)

    pltpu.sync_copy(sqp, out_hbm.at[pl.ds(base * L, C * L)])
    pltpu.sync_copy(sqn, out_hbm.at[pl.ds(B * L + base * L, C * L)])


_sc_sq = functools.partial(
    pl.kernel,
    out_type=jax.ShapeDtypeStruct((2 * B * L,), jnp.float32),
    mesh=_mesh,
    scratch_types=(
        [pltpu.VMEM((C,), jnp.int32) for _ in range(2)]
        + [pltpu.VMEM((C,), jnp.int32) for _ in range(2)]
        + [pltpu.VMEM((TCH, 128), jnp.float32) for _ in range(2)]
        + [pltpu.VMEM((16, 128), jnp.float32),
           pltpu.VMEM((C * D,), jnp.float32),
           pltpu.VMEM((C * D,), jnp.float32),
           pltpu.VMEM((C * L,), jnp.float32),
           pltpu.VMEM((C * L,), jnp.float32),
           pltpu.SemaphoreType.DMA]
    ),
)(_sq_body)

_W = 32256
_NBLK = NMAIN // _W


def _t_body(x_ref, o_ref):
    y = jnp.transpose(x_ref[...])
    y4 = y.reshape(_W // 512, 4, 128, D)
    for q in range(4):
        o_ref[:, q * D:(q + 1) * D] = y4[:, q].reshape(_W // 4, D)


_to_rows = pl.pallas_call(
    _t_body,
    grid=(_NBLK,),
    in_specs=[pl.BlockSpec((32, _W), lambda j: (0, j))],
    out_specs=pl.BlockSpec((_W // 4, 128), lambda j: (j, 0)),
    out_shape=jax.ShapeDtypeStruct((GMAIN, 128), jnp.float32),
)

_ROWS = 2 * B * L // 128
_HALF = _ROWS // 2


def _finish_body(x_ref, o_ref):
    x = x_ref[...]
    i128 = lax.broadcasted_iota(jnp.int32, (128, 8), 0)
    i8 = lax.broadcasted_iota(jnp.int32, (128, 8), 1)
    s_mat = jnp.where(i128 // L == i8, 1.0, 0.0).astype(jnp.float32)
    d2p = jnp.dot(x[:_HALF], s_mat, preferred_element_type=jnp.float32)
    d2n = jnp.dot(x[_HALF:], s_mat, preferred_element_type=jnp.float32)
    m = jnp.sqrt(d2p) - jnp.sqrt(d2n) + MARGIN
    o_ref[...] = jnp.sum(jnp.maximum(m, 0.0), keepdims=True) * (1.0 / B)


_finish = pl.pallas_call(
    _finish_body,
    out_shape=jax.ShapeDtypeStruct((1, 1), jnp.float32),
)


def kernel(pos_x, neg_x, entity_weight, relation_weight):
    pos = pos_x.astype(jnp.int32)
    neg = neg_x.astype(jnp.int32)
    idx_flat = jnp.concatenate([
        pos[:, 0], pos[:, 2], pos[:, 1],
        neg[:, 0], neg[:, 2], neg[:, 1],
    ])
    ent4 = _to_rows(entity_weight.T)
    etail = entity_weight[NMAIN:].reshape(16, 128)
    u = _sc_u(ent4, etail, idx_flat)
    rel4 = _to_rows(relation_weight.T)
    rtail = relation_weight[NMAIN:].reshape(16, 128)
    sq = _sc_sq(rel4, rtail, idx_flat, u)
    return _finish(sq.reshape(_ROWS, 128))[0, 0]
